# baseline (speedup 1.0000x reference)
import jax
import jax.numpy as jnp
from jax import lax
from jax.experimental import pallas as pl

S = 2
BM = 2048


def _matmul_block(*refs):
    xt_refs = refs[:S]
    wt_ref = refs[S]
    o_ref = refs[S + 1]
    for s in range(S):
        o_ref[s * BM:(s + 1) * BM, :] = lax.dot_general(
            xt_refs[s][...], wt_ref[...],
            dimension_numbers=(((0,), (0,)), ((), ())),
            preferred_element_type=jnp.float32,
        )


@jax.jit
def kernel(x, W):
    B, K = x.shape
    N = W.shape[0]
    xt = x.T
    wt = W.T
    grid = (B // (S * BM),)
    x_specs = [
        pl.BlockSpec((K, BM), lambda i, s=s: (0, S * i + s)) for s in range(S)
    ]
    return pl.pallas_call(
        _matmul_block,
        grid=grid,
        in_specs=x_specs + [pl.BlockSpec((K, N), lambda i: (0, 0))],
        out_specs=pl.BlockSpec((S * BM, N), lambda i: (i, 0)),
        out_shape=jax.ShapeDtypeStruct((B, N), jnp.float32),
    )(*([xt] * S), wt)


# speedup vs baseline: 1.0139x; 1.0139x over previous
import jax
import jax.numpy as jnp
from jax import lax
from jax.experimental import pallas as pl

BM = 2048


def _matmul_block(xt_ref, wt_ref, o_ref):
    o_ref[...] = lax.dot_general(
        xt_ref[...], wt_ref[...],
        dimension_numbers=(((0,), (0,)), ((), ())),
        preferred_element_type=jnp.float32,
    )


@jax.jit
def kernel(x, W):
    B, K = x.shape
    N = W.shape[0]
    xt = x.T
    wt = W.T
    grid = (B // BM,)
    return pl.pallas_call(
        _matmul_block,
        grid=grid,
        in_specs=[
            pl.BlockSpec((K, BM), lambda i: (0, i)),
            pl.BlockSpec((K, N), lambda i: (0, 0)),
        ],
        out_specs=pl.BlockSpec((BM, N), lambda i: (i, 0)),
        out_shape=jax.ShapeDtypeStruct((B, N), jnp.float32),
    )(xt, wt)
